# baseline (speedup 1.0000x reference)
import functools

import numpy as np
import jax
import jax.numpy as jnp
from jax import lax
from jax.experimental import pallas as pl
from jax.experimental.pallas import tpu as pltpu
from jax.experimental.pallas import tpu_sc as plsc

N = 10000
E = 320000
D_IN = 128
D_HID = 128
D_OUT = 64
K = 10

NPAD = 10240
NC, NS = 2, 16
NW = NC * NS
EPAD = 327680
EPT = EPAD // NW
SUB = 128
NSUB = EPT // SUB
RPT = NPAD // NS
DPT = NPAD // NS


def _jackson_damp():
    k = np.arange(K + 1, dtype=np.float64)
    c = np.pi / (K + 2)
    damp = ((K + 2 - k) * np.sin(c) * np.cos(k * c)
            + np.cos(c) * np.sin(k * c)) / ((K + 2) * np.sin(c))
    return damp.astype(np.float32)


_DAMP = _jackson_damp()


def _mlp_body(x_ref, w1t_ref, b1_ref, w2t_ref, b2_ref, h_ref):
    h1 = jnp.dot(x_ref[...], w1t_ref[...], preferred_element_type=jnp.float32)
    h1 = h1 + b1_ref[...][None, :]
    h1 = jnp.where(h1 > 0, h1, 0.01 * h1)
    h2 = jnp.dot(h1, w2t_ref[...], preferred_element_type=jnp.float32)
    h_ref[...] = h2 + b2_ref[...][None, :]


def _mlp(x, w1t, b1, w2t, b2):
    R = 1024
    return pl.pallas_call(
        _mlp_body,
        grid=(NPAD // R,),
        in_specs=[
            pl.BlockSpec((R, D_IN), lambda i: (i, 0)),
            pl.BlockSpec((D_IN, D_HID), lambda i: (0, 0)),
            pl.BlockSpec((D_HID,), lambda i: (0,)),
            pl.BlockSpec((D_HID, D_OUT), lambda i: (0, 0)),
            pl.BlockSpec((D_OUT,), lambda i: (0,)),
        ],
        out_specs=pl.BlockSpec((R, D_OUT), lambda i: (i, 0)),
        out_shape=jax.ShapeDtypeStruct((NPAD, D_OUT), jnp.float32),
    )(x, w1t, b1, w2t, b2)


def _deg_body(col_hbm, w_hbm, z_hbm, degp_hbm, colv, wv, deg_sh, ssem):
    c = lax.axis_index("c")
    s = lax.axis_index("s")
    wid = c * NS + s
    eb = wid * NSUB
    pltpu.sync_copy(col_hbm.at[pl.ds(eb, NSUB)], colv)
    pltpu.sync_copy(w_hbm.at[pl.ds(eb, NSUB)], wv)
    pltpu.sync_copy(z_hbm.at[pl.ds(s * DPT, DPT)], deg_sh.at[pl.ds(s * DPT, DPT)])
    plsc.subcore_barrier()

    for k in range(4):
        pltpu.async_copy(wv.at[k], deg_sh.at[colv.at[k]], ssem, add=True)

    def chunk(k, carry):
        pltpu.async_copy(wv.at[k], deg_sh.at[colv.at[k]], ssem, add=True)
        pltpu.make_async_copy(wv.at[0], deg_sh.at[colv.at[0]], ssem).wait()
        return carry

    lax.fori_loop(4, NSUB, chunk, 0)
    for k in range(4):
        pltpu.make_async_copy(wv.at[0], deg_sh.at[colv.at[0]], ssem).wait()
    plsc.subcore_barrier()
    pltpu.sync_copy(deg_sh.at[pl.ds(s * DPT, DPT)],
                    degp_hbm.at[c, pl.ds(s * DPT, DPT)])


def _sc_params():
    return pltpu.CompilerParams(needs_layout_passes=False, use_tc_tiling_on_sc=False)


def _deg(col, w, zpad):
    mesh = plsc.VectorSubcoreMesh(core_axis_name="c", subcore_axis_name="s")
    f = pl.kernel(
        _deg_body,
        out_type=jax.ShapeDtypeStruct((NC, NPAD), jnp.float32),
        mesh=mesh,
        compiler_params=_sc_params(),
        scratch_types=[
            pltpu.VMEM((NSUB, SUB), jnp.int32),
            pltpu.VMEM((NSUB, SUB), jnp.float32),
            pltpu.VMEM_SHARED((NPAD,), jnp.float32),
            pltpu.SemaphoreType.DMA,
        ],
    )
    return f(col, w, zpad)


def _wexp_body(w_ref, o_ref):
    o_ref[...] = jnp.broadcast_to(w_ref[...], o_ref.shape)


def _wexp(w):
    R = 8192
    return pl.pallas_call(
        _wexp_body,
        grid=(EPAD // R,),
        in_specs=[pl.BlockSpec((R, 1), lambda i: (i, 0))],
        out_specs=pl.BlockSpec((R, 16), lambda i: (i, 0)),
        out_shape=jax.ShapeDtypeStruct((EPAD, 16), jnp.float32),
    )(w.reshape(EPAD, 1))


def _hop_body(sv_hbm, row_hbm, col_hbm, w_hbm, z_hbm, acc_hbm,
              rowv, colv, w0, w1, w2, w3, w4, b0, b1, b2, b3, b4, acc_sh,
              g0, g1, g2, g3, g4, s0, s1, s2, s3, s4,
              m0, m1, m2, m3, m4):
    c = lax.axis_index("c")
    s = lax.axis_index("s")
    wid = c * NS + s
    eb = wid * NSUB
    pltpu.sync_copy(row_hbm.at[pl.ds(eb, NSUB)], rowv)
    pltpu.sync_copy(col_hbm.at[pl.ds(eb, NSUB)], colv)
    pltpu.sync_copy(z_hbm.at[pl.ds(s * RPT, RPT)], acc_sh.at[pl.ds(s * RPT, RPT)])
    plsc.subcore_barrier()

    bufs = (b0, b1, b2, b3, b4)
    wbufs = (w0, w1, w2, w3, w4)
    gsems = (g0, g1, g2, g3, g4)
    ssems = (s0, s1, s2, s3, s4)
    wsems = (m0, m1, m2, m3, m4)

    def scale(bi, k):
        buf = bufs[bi]
        wb = wbufs[bi]

        def grp(g, carry):
            for i in range(16):
                e = g * 16 + i
                bwi = wb[e, pl.ds(0, 16)]
                for q in range(4):
                    buf[e, pl.ds(q * 16, 16)] = buf[e, pl.ds(q * 16, 16)] * bwi
            return carry

        lax.fori_loop(0, SUB // 16, grp, 0)

    def gissue(a, bi):
        pltpu.async_copy(sv_hbm.at[rowv.at[a]], bufs[bi], gsems[bi])
        pltpu.async_copy(w_hbm.at[pl.ds(wid * EPT + a * SUB, SUB)], wbufs[bi],
                         wsems[bi])

    def gwait(bi):
        pltpu.make_async_copy(sv_hbm.at[rowv.at[0]], bufs[bi], gsems[bi]).wait()
        pltpu.make_async_copy(w_hbm.at[pl.ds(0, SUB)], wbufs[bi],
                              wsems[bi]).wait()

    def sissue(a, bi):
        pltpu.async_copy(bufs[bi], acc_sh.at[colv.at[a]], ssems[bi], add=True)

    def swait(bi):
        pltpu.make_async_copy(bufs[bi], acc_sh.at[colv.at[0]], ssems[bi]).wait()

    gissue(0, 0)
    gissue(1, 1)
    for a in range(3):
        gwait(a)
        scale(a, a)
        gissue(a + 2, (a + 2) % 5)
        sissue(a, a)

    def body(kk, carry):
        for jj in range(5):
            a = 3 + 5 * kk + jj
            bi = (3 + jj) % 5
            gwait(bi)
            scale(bi, a)
            swait((bi + 2) % 5)
            gissue(a + 2, (bi + 2) % 5)
            sissue(a, bi)
        return carry

    lax.fori_loop(0, (NSUB - 5) // 5, body, 0)
    for a in (NSUB - 2, NSUB - 1):
        bi = a % 5
        gwait(bi)
        scale(bi, a)
        swait((bi + 2) % 5)
        sissue(a, bi)
    for bi in (2, 3, 4):
        swait(bi)
    plsc.subcore_barrier()
    pltpu.sync_copy(acc_sh.at[pl.ds(s * RPT, RPT)],
                    acc_hbm.at[c, pl.ds(s * RPT, RPT)])


def _hop(sv, row, col, w, z2):
    mesh = plsc.VectorSubcoreMesh(core_axis_name="c", subcore_axis_name="s")
    f = pl.kernel(
        _hop_body,
        out_type=jax.ShapeDtypeStruct((NC, NPAD, D_OUT), jnp.float32),
        mesh=mesh,
        compiler_params=_sc_params(),
        scratch_types=(
            [pltpu.VMEM((NSUB, SUB), jnp.int32),
             pltpu.VMEM((NSUB, SUB), jnp.int32)]
            + [pltpu.VMEM((SUB, 16), jnp.float32)] * 5
            + [pltpu.VMEM((SUB, D_OUT), jnp.float32)] * 5
            + [pltpu.VMEM_SHARED((NPAD, D_OUT), jnp.float32)]
            + [pltpu.SemaphoreType.DMA] * 15
        ),
    )
    return f(sv, row, col, w, z2)


def _prep_body(degp_ref, h_ref, dinv_ref, sv_ref):
    deg = degp_ref[0, :] + degp_ref[1, :]
    dinv = jnp.where(deg > 0, lax.rsqrt(jnp.maximum(deg, 1e-12)), 0.0)
    dinv_ref[...] = dinv[:, None]
    sv_ref[...] = h_ref[...] * dinv[:, None]


def _prep(degp, h):
    R = 1024
    return pl.pallas_call(
        _prep_body,
        grid=(NPAD // R,),
        in_specs=[
            pl.BlockSpec((NC, R), lambda i: (0, i)),
            pl.BlockSpec((R, D_OUT), lambda i: (i, 0)),
        ],
        out_specs=[
            pl.BlockSpec((R, 1), lambda i: (i, 0)),
            pl.BlockSpec((R, D_OUT), lambda i: (i, 0)),
        ],
        out_shape=[
            jax.ShapeDtypeStruct((NPAD, 1), jnp.float32),
            jax.ShapeDtypeStruct((NPAD, D_OUT), jnp.float32),
        ],
    )(degp, h)


def _comb1_body(acc_ref, dinv_ref, h_ref, c01_ref, tx1_ref, sv_ref, out_ref):
    p = (acc_ref[0] + acc_ref[1]) * dinv_ref[...]
    tx1_ref[...] = p
    sv_ref[...] = p * dinv_ref[...]
    out_ref[...] = c01_ref[0, 0] * h_ref[...] + c01_ref[0, 1] * p


def _comb1(acc, dinv, h, c01):
    R = 1024
    return pl.pallas_call(
        _comb1_body,
        grid=(NPAD // R,),
        in_specs=[
            pl.BlockSpec((NC, R, D_OUT), lambda i: (0, i, 0)),
            pl.BlockSpec((R, 1), lambda i: (i, 0)),
            pl.BlockSpec((R, D_OUT), lambda i: (i, 0)),
            pl.BlockSpec(memory_space=pltpu.SMEM),
        ],
        out_specs=[
            pl.BlockSpec((R, D_OUT), lambda i: (i, 0)),
            pl.BlockSpec((R, D_OUT), lambda i: (i, 0)),
            pl.BlockSpec((R, D_OUT), lambda i: (i, 0)),
        ],
        out_shape=[
            jax.ShapeDtypeStruct((NPAD, D_OUT), jnp.float32),
            jax.ShapeDtypeStruct((NPAD, D_OUT), jnp.float32),
            jax.ShapeDtypeStruct((NPAD, D_OUT), jnp.float32),
        ],
    )(acc, dinv, h, c01)


def _comb2_body(acc_ref, dinv_ref, tx0_ref, outp_ref, ck_ref,
                tx2_ref, sv_ref, out_ref):
    p = (acc_ref[0] + acc_ref[1]) * dinv_ref[...]
    t2 = 2.0 * p - tx0_ref[...]
    tx2_ref[...] = t2
    sv_ref[...] = t2 * dinv_ref[...]
    out_ref[...] = outp_ref[...] + ck_ref[0, 0] * t2


def _comb2(acc, dinv, tx0, outp, ck):
    R = 1024
    return pl.pallas_call(
        _comb2_body,
        grid=(NPAD // R,),
        in_specs=[
            pl.BlockSpec((NC, R, D_OUT), lambda i: (0, i, 0)),
            pl.BlockSpec((R, 1), lambda i: (i, 0)),
            pl.BlockSpec((R, D_OUT), lambda i: (i, 0)),
            pl.BlockSpec((R, D_OUT), lambda i: (i, 0)),
            pl.BlockSpec(memory_space=pltpu.SMEM),
        ],
        out_specs=[
            pl.BlockSpec((R, D_OUT), lambda i: (i, 0)),
            pl.BlockSpec((R, D_OUT), lambda i: (i, 0)),
            pl.BlockSpec((R, D_OUT), lambda i: (i, 0)),
        ],
        out_shape=[
            jax.ShapeDtypeStruct((NPAD, D_OUT), jnp.float32),
            jax.ShapeDtypeStruct((NPAD, D_OUT), jnp.float32),
            jax.ShapeDtypeStruct((NPAD, D_OUT), jnp.float32),
        ],
    )(acc, dinv, tx0, outp, ck)


def kernel(x, edge_index, edge_weight, W1, b1, W2, b2, cheb_coef):
    pad = EPAD - E
    rowp = jnp.concatenate([edge_index[0], jnp.zeros((pad,), jnp.int32)])
    colp = jnp.concatenate([edge_index[1], jnp.zeros((pad,), jnp.int32)])
    ewp = jnp.concatenate([edge_weight, jnp.zeros((pad,), jnp.float32)])
    row = rowp.reshape(EPAD // SUB, SUB)
    col = colp.reshape(EPAD // SUB, SUB)
    ew = ewp.reshape(EPAD // SUB, SUB)
    wexp = _wexp(ewp)
    h = _mlp(x, W1.T, b1, W2.T, b2)

    zpad = jnp.zeros((NPAD,), jnp.float32)
    z2 = jnp.zeros((NPAD, D_OUT), jnp.float32)
    degp = _deg(col, ew, zpad)
    dinv, sv = _prep(degp, h)

    coefs = cheb_coef * jnp.asarray(_DAMP)

    acc = _hop(sv, row, col, wexp, z2)
    tx1, sv, out = _comb1(acc, dinv, h, coefs[0:2].reshape(1, 2))
    tx0 = h
    for k in range(2, K + 1):
        acc = _hop(sv, row, col, wexp, z2)
        tx2, sv, out = _comb2(acc, dinv, tx0, out, coefs[k].reshape(1, 1))
        tx0, tx1 = tx1, tx2
    return out[:N]
